# baseline (speedup 1.0000x reference)
import functools

import numpy as np
import jax
import jax.numpy as jnp
from jax.experimental import pallas as pl
from jax.experimental.pallas import tpu as pltpu


def _fused_kernel(xt_ref, w1a_ref, w2_ref, b2_ref, o_ref, *, nb):
    L = xt_ref.shape[2]
    for b in range(nb):
        xt = xt_ref[b].astype(jnp.bfloat16)
        ones = jnp.ones((4, L), jnp.bfloat16)
        xa = jnp.concatenate([xt, ones], axis=0)
        h = jax.lax.dot_general(xa, w1a_ref[...],
                                (((0,), (0,)), ((), ())),
                                preferred_element_type=jnp.float32)
        hb = h.astype(jnp.bfloat16)
        hb = jnp.maximum(hb, jnp.bfloat16(0.1) * hb)
        acc = jnp.dot(hb, w2_ref[...], preferred_element_type=jnp.float32)
        o_ref[b] = acc + b2_ref[...]


def _mlp_forward(x, w1, b1, w2, b2):
    B, L, n_input = x.shape
    n_hidden = w1.shape[1]
    emb = w2.shape[1]
    rows = B * L

    xt = jnp.transpose(x, (0, 2, 1))

    w1a = jnp.concatenate(
        [w1, b1, jnp.zeros((3, n_hidden), w1.dtype)], axis=0)
    w1a = w1a.astype(jnp.bfloat16)
    w2b = w2.astype(jnp.bfloat16)

    nb = 8
    while B % nb:
        nb //= 2
    tl = 2048
    while L % tl:
        tl //= 2
    grid = (B // nb, L // tl)
    out = pl.pallas_call(
        functools.partial(_fused_kernel, nb=nb),
        out_shape=jax.ShapeDtypeStruct((B, L, emb), jnp.float32),
        grid=grid,
        in_specs=[
            pl.BlockSpec((nb, n_input, tl), lambda i, j: (i, 0, j)),
            pl.BlockSpec(w1a.shape, lambda i, j: (0, 0)),
            pl.BlockSpec(w2b.shape, lambda i, j: (0, 0)),
            pl.BlockSpec(b2.shape, lambda i, j: (0, 0)),
        ],
        out_specs=pl.BlockSpec((nb, tl, emb), lambda i, j: (i, j, 0)),
        compiler_params=pltpu.CompilerParams(
            dimension_semantics=("parallel", "parallel")),
    )(xt, w1a, w2b, b2)

    return out


def kernel(x, w1, b1, w2, b2):
    devs = jax.devices()
    if len(devs) < 2 or x.shape[0] % 2:
        return _mlp_forward(x, w1, b1, w2, b2)
    mesh = jax.sharding.Mesh(np.asarray(devs[:2]), ("d",))
    P = jax.sharding.PartitionSpec
    f = jax.shard_map(_mlp_forward, mesh=mesh,
                      in_specs=(P("d"), P(), P(), P(), P()),
                      out_specs=P("d"), check_vma=False)
    return f(x, w1, b1, w2, b2)


# speedup vs baseline: 1.0916x; 1.0916x over previous
import functools

import numpy as np
import jax
import jax.numpy as jnp
from jax.experimental import pallas as pl
from jax.experimental.pallas import tpu as pltpu


def _fused_kernel(xt_ref, w1a_ref, w2_ref, b2_ref, o_ref, *, nb):
    L = xt_ref.shape[2]
    for b in range(nb):
        xt = xt_ref[b].astype(jnp.bfloat16)
        ones = jnp.ones((4, L), jnp.bfloat16)
        xa = jnp.concatenate([xt, ones], axis=0)
        h = jax.lax.dot_general(xa, w1a_ref[...],
                                (((0,), (0,)), ((), ())),
                                preferred_element_type=jnp.float32)
        hb = h.astype(jnp.bfloat16)
        hb = jnp.maximum(hb, jnp.bfloat16(0.1) * hb)
        acc = jnp.dot(hb, w2_ref[...], preferred_element_type=jnp.float32)
        o_ref[b] = acc + b2_ref[...]


def _mlp_forward(x, w1, b1, w2, b2):
    B, L, n_input = x.shape
    n_hidden = w1.shape[1]
    emb = w2.shape[1]
    rows = B * L

    xt = jnp.transpose(x, (0, 2, 1))

    w1a = jnp.concatenate(
        [w1, b1, jnp.zeros((3, n_hidden), w1.dtype)], axis=0)
    w1a = w1a.astype(jnp.bfloat16)
    w2b = w2.astype(jnp.bfloat16)

    nb = 16
    while B % nb:
        nb //= 2
    tl = 2048
    while L % tl:
        tl //= 2
    grid = (B // nb, L // tl)
    out = pl.pallas_call(
        functools.partial(_fused_kernel, nb=nb),
        out_shape=jax.ShapeDtypeStruct((B, L, emb), jnp.float32),
        grid=grid,
        in_specs=[
            pl.BlockSpec((nb, n_input, tl), lambda i, j: (i, 0, j)),
            pl.BlockSpec(w1a.shape, lambda i, j: (0, 0)),
            pl.BlockSpec(w2b.shape, lambda i, j: (0, 0)),
            pl.BlockSpec(b2.shape, lambda i, j: (0, 0)),
        ],
        out_specs=pl.BlockSpec((nb, tl, emb), lambda i, j: (i, j, 0)),
        compiler_params=pltpu.CompilerParams(
            dimension_semantics=("parallel", "parallel")),
    )(xt, w1a, w2b, b2)

    return out


def kernel(x, w1, b1, w2, b2):
    devs = jax.devices()
    if len(devs) < 2 or x.shape[0] % 2:
        return _mlp_forward(x, w1, b1, w2, b2)
    mesh = jax.sharding.Mesh(np.asarray(devs[:2]), ("d",))
    P = jax.sharding.PartitionSpec
    f = jax.shard_map(_mlp_forward, mesh=mesh,
                      in_specs=(P("d"), P(), P(), P(), P()),
                      out_specs=P("d"), check_vma=False)
    return f(x, w1, b1, w2, b2)
